# baseline (speedup 1.0000x reference)
import functools

import jax
import jax.numpy as jnp
from jax import lax
from jax.experimental import pallas as pl
from jax.experimental.pallas import tpu as pltpu
from jax.experimental.pallas import tpu_sc as plsc
import numpy as np

_D = 256
_H = 8
_L = 3
_P = 4
_DH = 32
_SPATIAL = [(80, 80), (40, 40), (20, 20)]
_START = [0, 6400, 8000]
_B = 16
_Q = 300
_S = 8400

_ROWS = _B * _Q
_NLP = _L * _P
_LANES = _NLP * _H
_NCOL = 4 * _LANES
_TROWS = _B * _S * _H


_VT = 1200


def _vproj_body(x_ref, w_ref, b_ref, o_ref):
    o_ref[...] = jnp.dot(x_ref[...], w_ref[...],
                         preferred_element_type=jnp.float32) + b_ref[...]


def _vproj_pack_body(x_ref, w_ref, b_ref, o_ref):
    y = jnp.dot(x_ref[...], w_ref[...],
                preferred_element_type=jnp.float32) + b_ref[...]
    o_ref[...] = y.astype(jnp.bfloat16)


def _value_projection(enc2, W_val, b_val):
    n = enc2.shape[0]
    return pl.pallas_call(
        _vproj_pack_body,
        grid=(n // _VT,),
        in_specs=[
            pl.BlockSpec((_VT, _D), lambda i: (i, 0)),
            pl.BlockSpec((_D, _D), lambda i: (0, 0)),
            pl.BlockSpec((1, _D), lambda i: (0, 0)),
        ],
        out_specs=pl.BlockSpec((_VT, _D), lambda i: (i, 0)),
        out_shape=jax.ShapeDtypeStruct((n, _D), jnp.bfloat16),
    )(enc2, W_val, b_val.reshape(1, _D))


_RT = 600


def _sample_body(hs_ref, rp_ref, wcat_ref, bcat_ref, g_ref, gt_ref,
                 idx_ref, w_ref):
    i = pl.program_id(0)
    o = jnp.dot(hs_ref[...], wcat_ref[...],
                preferred_element_type=jnp.float32) + bcat_ref[...]
    ox = o[:, 0:_LANES]
    oy = o[:, _LANES:2 * _LANES]
    a = o[:, 2 * _LANES:3 * _LANES]

    lam = lax.broadcasted_iota(jnp.int32, (_RT, _LANES), 1)
    h_i = lam % _H
    l_i = (lam // _H) // _P
    wl_i = jnp.where(l_i == 0, 80, jnp.where(l_i == 1, 40, 20))
    start_i = jnp.where(l_i == 0, 0, jnp.where(l_i == 1, 6400, 8000))
    wl_f = wl_i.astype(jnp.float32)

    row_i = lax.broadcasted_iota(jnp.int32, (_RT, _LANES), 0) + i * _RT
    b_i = row_i // _Q

    rx = rp_ref[:, 0:1]
    ry = rp_ref[:, 1:2]
    rw = rp_ref[:, 2:3]
    rh = rp_ref[:, 3:4]

    locx = rx + ox * (rw * (0.5 / _P))
    locy = ry + oy * (rh * (0.5 / _P))
    ix = locx * wl_f - 0.5
    iy = locy * wl_f - 0.5
    x0f = jnp.floor(ix)
    y0f = jnp.floor(iy)
    wx1 = ix - x0f
    wx0 = 1.0 - wx1
    wy1 = iy - y0f
    wy0 = 1.0 - wy1
    x0 = x0f.astype(jnp.int32)
    y0 = y0f.astype(jnp.int32)
    x1 = x0 + 1
    y1 = y0 + 1

    m = jnp.max(a, axis=1, keepdims=True)
    e = jnp.exp(a - m)
    ssum = jnp.dot(e, g_ref[...], preferred_element_type=jnp.float32)
    denom = jnp.dot(ssum, gt_ref[...], preferred_element_type=jnp.float32)
    attn = e / denom

    base_i = b_i * _S + start_i

    def corner(xc, yc, wx, wy):
        vx = (xc >= 0) & (xc < wl_i)
        vy = (yc >= 0) & (yc < wl_i)
        xcl = jnp.clip(xc, 0, wl_i - 1)
        ycl = jnp.clip(yc, 0, wl_i - 1)
        idx = (base_i + ycl * wl_i + xcl) * _H + h_i
        w = attn * wx * wy * jnp.where(vx & vy, 1.0, 0.0)
        return idx, w

    i00, w00 = corner(x0, y0, wx0, wy0)
    i10, w10 = corner(x1, y0, wx1, wy0)
    i01, w01 = corner(x0, y1, wx0, wy1)
    i11, w11 = corner(x1, y1, wx1, wy1)

    idx_ref[...] = jnp.concatenate([i00, i10, i01, i11], axis=1)
    w_ref[...] = jnp.concatenate([w00, w10, w01, w11], axis=1)


def _sampling_params(hs2, rp2, W_cat, b_cat, G, GT):
    return pl.pallas_call(
        _sample_body,
        grid=(_ROWS // _RT,),
        in_specs=[
            pl.BlockSpec((_RT, _D), lambda i: (i, 0)),
            pl.BlockSpec((_RT, 4), lambda i: (i, 0)),
            pl.BlockSpec((_D, 3 * _LANES), lambda i: (0, 0)),
            pl.BlockSpec((1, 3 * _LANES), lambda i: (0, 0)),
            pl.BlockSpec((_LANES, _H), lambda i: (0, 0)),
            pl.BlockSpec((_H, _LANES), lambda i: (0, 0)),
        ],
        out_specs=[
            pl.BlockSpec((_RT, _NCOL), lambda i: (i, 0)),
            pl.BlockSpec((_RT, _NCOL), lambda i: (i, 0)),
        ],
        out_shape=[
            jax.ShapeDtypeStruct((_ROWS, _NCOL), jnp.int32),
            jax.ShapeDtypeStruct((_ROWS, _NCOL), jnp.float32),
        ],
    )(hs2, rp2, W_cat, b_cat, G, GT)


_NW = 32
_RC = 8
_NCHUNKS = _ROWS // _RC
_KMAX = (_NCHUNKS + _NW - 1) // _NW
_NG = _RC * _NCOL // 128
_NVREG = _NCOL // 16


def _sc_body(table, idx, ww, out, idx_v, w_v, rows_v, out_v, sem):
    cid = lax.axis_index("c")
    sid = lax.axis_index("s")
    wid = sid * 2 + cid
    lane = lax.broadcasted_iota(jnp.int32, (16,), 0)
    lane8 = lane % _H
    lo_mask = lane < _H
    hi_mask = lane >= _H
    colbase = lane8 * _DH

    def chunk_body(k, _):
        ci = k * _NW + wid

        @pl.when(ci < _NCHUNKS)
        def _go():
            base_row = ci * _RC
            pltpu.sync_copy(idx.at[pl.ds(base_row, _RC)], idx_v)
            pltpu.sync_copy(ww.at[pl.ds(base_row, _RC)], w_v)
            for g in range(_NG):
                pltpu.async_copy(
                    table.at[idx_v.at[g // 3, pl.ds((g % 3) * 128, 128)]],
                    rows_v.at[pl.ds(g * 128, 128)], sem.at[g // 3])

            def row_body(rloc, _2):
                rb = rloc * _NCOL
                pltpu.make_async_copy(
                    table.at[pl.ds(0, _NCOL)],
                    rows_v.at[pl.ds(rloc * _NCOL, _NCOL)],
                    sem.at[rloc]).wait()
                rsp = jnp.zeros((16,), jnp.int32) + rloc
                for half in range(2):
                    accs = [jnp.zeros((16,), jnp.float32)] * 16
                    for g in range(_NVREG):
                        wv = w_v[rloc, pl.ds(g * 16, 16)]
                        ridx = lane + (rb + g * 16)
                        for wd in range(8):
                            word = half * 8 + wd
                            vals = plsc.load_gather(
                                rows_v,
                                [ridx, jnp.full((16,), word, jnp.int32)])
                            bc = plsc.bitcast(vals, jnp.bfloat16)
                            lo, hi = plsc.unpack(
                                bc, format=plsc.PackFormat.INTERLEAVED,
                                preferred_element_type=jnp.float32)
                            accs[2 * wd] = accs[2 * wd] + wv * lo
                            accs[2 * wd + 1] = accs[2 * wd + 1] + wv * hi
                    for c in range(16):
                        ch = half * 16 + c
                        cc = colbase + ch
                        plsc.store_scatter(out_v, [rsp, cc], accs[c],
                                           mask=lo_mask)
                        plsc.addupdate_scatter(out_v, [rsp, cc], accs[c],
                                               mask=hi_mask)
                return 0

            lax.fori_loop(0, _RC, row_body, 0)
            pltpu.sync_copy(out_v, out.at[pl.ds(base_row, _RC)])
        return 0

    lax.fori_loop(0, _KMAX, chunk_body, 0)


def _sc_gather_reduce(table, idx, ww):
    mesh = plsc.VectorSubcoreMesh(core_axis_name="c", subcore_axis_name="s")
    f = pl.kernel(
        _sc_body,
        out_type=jax.ShapeDtypeStruct((_ROWS, _D), jnp.float32),
        mesh=mesh,
        scratch_types=[
            pltpu.VMEM((_RC, _NCOL), jnp.int32),
            pltpu.VMEM((_RC, _NCOL), jnp.float32),
            pltpu.VMEM((_NG * 128, _DH // 2), jnp.int32),
            pltpu.VMEM((_RC, _D), jnp.float32),
            pltpu.SemaphoreType.DMA((_RC,)),
        ],
        compiler_params=pltpu.CompilerParams(needs_layout_passes=False,
                                             use_tc_tiling_on_sc=False),
    )
    return f(table, idx, ww)


def _out_projection(x2, W_out, b_out):
    return pl.pallas_call(
        _vproj_body,
        grid=(_ROWS // _RT,),
        in_specs=[
            pl.BlockSpec((_RT, _D), lambda i: (i, 0)),
            pl.BlockSpec((_D, _D), lambda i: (0, 0)),
            pl.BlockSpec((1, _D), lambda i: (0, 0)),
        ],
        out_specs=pl.BlockSpec((_RT, _D), lambda i: (i, 0)),
        out_shape=jax.ShapeDtypeStruct((_ROWS, _D), jnp.float32),
    )(x2, W_out, b_out.reshape(1, _D))


def _perms():
    lam = np.arange(_LANES)
    h = lam % _H
    lp = lam // _H
    l = lp // _P
    p = lp % _P
    base = (h * _L + l) * _P + p
    perm_x = base * 2
    perm_y = base * 2 + 1
    perm_a = base
    return perm_x, perm_y, perm_a


_PERM_X, _PERM_Y, _PERM_A = _perms()
_G_NP = (np.arange(_LANES)[:, None] % _H == np.arange(_H)[None, :])
_G_NP = _G_NP.astype(np.float32)


def kernel(hidden_states, encoder_hidden_states, reference_points,
           W_off, b_off, W_attn, b_attn, W_val, b_val, W_out, b_out):
    enc2 = encoder_hidden_states.reshape(_B * _S, _D)
    hs2 = hidden_states.reshape(_ROWS, _D)
    rp2 = reference_points.reshape(_ROWS, 4)

    W_cat = jnp.concatenate(
        [W_off[:, _PERM_X], W_off[:, _PERM_Y], W_attn[:, _PERM_A]], axis=1)
    b_cat = jnp.concatenate(
        [b_off[_PERM_X], b_off[_PERM_Y], b_attn[_PERM_A]]).reshape(1, -1)
    G = jnp.asarray(_G_NP)
    GT = jnp.asarray(_G_NP.T)

    value_bf = _value_projection(enc2, W_val, b_val)
    table = jax.lax.bitcast_convert_type(
        value_bf.reshape(_TROWS, _DH // 2, 2), jnp.int32)

    idx, ww = _sampling_params(hs2, rp2, W_cat, b_cat, G, GT)

    x2 = _sc_gather_reduce(table, idx, ww)

    out = _out_projection(x2, W_out, b_out)
    return out.reshape(_B, _Q, _D)


# speedup vs baseline: 25.0419x; 25.0419x over previous
import functools

import jax
import jax.numpy as jnp
from jax import lax
from jax.experimental import pallas as pl
from jax.experimental.pallas import tpu as pltpu
from jax.experimental.pallas import tpu_sc as plsc
import numpy as np

_D = 256
_H = 8
_L = 3
_P = 4
_DH = 32
_SPATIAL = [(80, 80), (40, 40), (20, 20)]
_START = [0, 6400, 8000]
_B = 16
_Q = 300
_S = 8400

_ROWS = _B * _Q
_NLP = _L * _P
_LANES = _NLP * _H
_NCOL = 4 * _LANES
_TROWS = _B * _S * _H


_VT = 1200


def _vproj_body(x_ref, w_ref, b_ref, o_ref):
    o_ref[...] = jnp.dot(x_ref[...], w_ref[...],
                         preferred_element_type=jnp.float32) + b_ref[...]


def _vproj_pack_body(x_ref, w_ref, b_ref, o_ref):
    y = jnp.dot(x_ref[...], w_ref[...],
                preferred_element_type=jnp.float32) + b_ref[...]
    o_ref[...] = y.astype(jnp.bfloat16)


def _value_projection(enc2, W_val, b_val):
    n = enc2.shape[0]
    return pl.pallas_call(
        _vproj_pack_body,
        grid=(n // _VT,),
        in_specs=[
            pl.BlockSpec((_VT, _D), lambda i: (i, 0)),
            pl.BlockSpec((_D, _D), lambda i: (0, 0)),
            pl.BlockSpec((1, _D), lambda i: (0, 0)),
        ],
        out_specs=pl.BlockSpec((_VT, _D), lambda i: (i, 0)),
        out_shape=jax.ShapeDtypeStruct((n, _D), jnp.bfloat16),
    )(enc2, W_val, b_val.reshape(1, _D))


_RT = 600


def _sample_body(hs_ref, rp_ref, wcat_ref, bcat_ref, g_ref, gt_ref,
                 idx_ref, w_ref):
    i = pl.program_id(0)
    o = jnp.dot(hs_ref[...], wcat_ref[...],
                preferred_element_type=jnp.float32) + bcat_ref[...]
    ox = o[:, 0:_LANES]
    oy = o[:, _LANES:2 * _LANES]
    a = o[:, 2 * _LANES:3 * _LANES]

    lam = lax.broadcasted_iota(jnp.int32, (_RT, _LANES), 1)
    h_i = lam % _H
    l_i = (lam // _H) // _P
    wl_i = jnp.where(l_i == 0, 80, jnp.where(l_i == 1, 40, 20))
    start_i = jnp.where(l_i == 0, 0, jnp.where(l_i == 1, 6400, 8000))
    wl_f = wl_i.astype(jnp.float32)

    row_i = lax.broadcasted_iota(jnp.int32, (_RT, _LANES), 0) + i * _RT
    b_i = row_i // _Q

    rx = rp_ref[:, 0:1]
    ry = rp_ref[:, 1:2]
    rw = rp_ref[:, 2:3]
    rh = rp_ref[:, 3:4]

    locx = rx + ox * (rw * (0.5 / _P))
    locy = ry + oy * (rh * (0.5 / _P))
    ix = locx * wl_f - 0.5
    iy = locy * wl_f - 0.5
    x0f = jnp.floor(ix)
    y0f = jnp.floor(iy)
    wx1 = ix - x0f
    wx0 = 1.0 - wx1
    wy1 = iy - y0f
    wy0 = 1.0 - wy1
    x0 = x0f.astype(jnp.int32)
    y0 = y0f.astype(jnp.int32)
    x1 = x0 + 1
    y1 = y0 + 1

    m = jnp.max(a, axis=1, keepdims=True)
    e = jnp.exp(a - m)
    ssum = jnp.dot(e, g_ref[...], preferred_element_type=jnp.float32)
    denom = jnp.dot(ssum, gt_ref[...], preferred_element_type=jnp.float32)
    attn = e / denom

    base_i = b_i * _S + start_i

    def corner(xc, yc, wx, wy):
        vx = (xc >= 0) & (xc < wl_i)
        vy = (yc >= 0) & (yc < wl_i)
        xcl = jnp.clip(xc, 0, wl_i - 1)
        ycl = jnp.clip(yc, 0, wl_i - 1)
        idx = (base_i + ycl * wl_i + xcl) * _H + h_i
        w = attn * wx * wy * jnp.where(vx & vy, 1.0, 0.0)
        return idx, w

    i00, w00 = corner(x0, y0, wx0, wy0)
    i10, w10 = corner(x1, y0, wx1, wy0)
    i01, w01 = corner(x0, y1, wx0, wy1)
    i11, w11 = corner(x1, y1, wx1, wy1)

    idx_ref[...] = jnp.concatenate([i00, i10, i01, i11], axis=1)
    w_ref[...] = jnp.concatenate([w00, w10, w01, w11], axis=1)


def _sampling_params(hs2, rp2, W_cat, b_cat, G, GT):
    return pl.pallas_call(
        _sample_body,
        grid=(_ROWS // _RT,),
        in_specs=[
            pl.BlockSpec((_RT, _D), lambda i: (i, 0)),
            pl.BlockSpec((_RT, 4), lambda i: (i, 0)),
            pl.BlockSpec((_D, 3 * _LANES), lambda i: (0, 0)),
            pl.BlockSpec((1, 3 * _LANES), lambda i: (0, 0)),
            pl.BlockSpec((_LANES, _H), lambda i: (0, 0)),
            pl.BlockSpec((_H, _LANES), lambda i: (0, 0)),
        ],
        out_specs=[
            pl.BlockSpec((_RT, _NCOL), lambda i: (i, 0)),
            pl.BlockSpec((_RT, _NCOL), lambda i: (i, 0)),
        ],
        out_shape=[
            jax.ShapeDtypeStruct((_ROWS, _NCOL), jnp.int32),
            jax.ShapeDtypeStruct((_ROWS, _NCOL), jnp.float32),
        ],
    )(hs2, rp2, W_cat, b_cat, G, GT)


_NW = 32
_RC = 8
_NCHUNKS = _ROWS // _RC
_KMAX = (_NCHUNKS + _NW - 1) // _NW
_NG = _RC * _NCOL // 128
_NVREG = _NCOL // 16


def _sc_body(table, idx, ww, out, idx_v, w_v, rows_v, out_v, sem):
    cid = lax.axis_index("c")
    sid = lax.axis_index("s")
    wid = sid * 2 + cid
    lane = lax.broadcasted_iota(jnp.int32, (16,), 0)
    lane8 = lane % _H
    lo_mask = lane < _H
    hi_mask = lane >= _H
    colbase = lane8 * _DH

    def chunk_body(k, _):
        ci = k * _NW + wid

        @pl.when(ci < _NCHUNKS)
        def _go():
            base_row = ci * _RC
            pltpu.sync_copy(idx.at[pl.ds(base_row, _RC)], idx_v)
            pltpu.sync_copy(ww.at[pl.ds(base_row, _RC)], w_v)
            for g in range(_NG):
                pltpu.async_copy(
                    table.at[idx_v.at[g // 3, pl.ds((g % 3) * 128, 128)]],
                    rows_v.at[pl.ds(g * 128, 128)], sem.at[g // 3])

            def row_body(rloc, _2):
                rb = rloc * _NCOL
                pltpu.make_async_copy(
                    table.at[pl.ds(0, _NCOL)],
                    rows_v.at[pl.ds(rloc * _NCOL, _NCOL)],
                    sem.at[rloc]).wait()
                rsp = jnp.zeros((16,), jnp.int32) + rloc
                for half in range(2):
                    accs = [jnp.zeros((16,), jnp.float32)] * 16
                    for g in range(_NVREG):
                        wv = w_v[rloc, pl.ds(g * 16, 16)]
                        ridx = lane + (rb + g * 16)
                        for wd in range(8):
                            word = half * 8 + wd
                            vals = plsc.load_gather(
                                rows_v,
                                [ridx, jnp.full((16,), word, jnp.int32)])
                            bc = plsc.bitcast(vals, jnp.bfloat16)
                            lo, hi = plsc.unpack(
                                bc, format=plsc.PackFormat.INTERLEAVED,
                                preferred_element_type=jnp.float32)
                            accs[2 * wd] = accs[2 * wd] + wv * lo
                            accs[2 * wd + 1] = accs[2 * wd + 1] + wv * hi
                    for c in range(16):
                        ch = half * 16 + c
                        cc = colbase + ch
                        plsc.store_scatter(out_v, [rsp, cc], accs[c],
                                           mask=lo_mask)
                        plsc.addupdate_scatter(out_v, [rsp, cc], accs[c],
                                               mask=hi_mask)
                return 0

            lax.fori_loop(0, _RC, row_body, 0)
            pltpu.sync_copy(out_v, out.at[pl.ds(base_row, _RC)])
        return 0

    lax.fori_loop(0, _KMAX, chunk_body, 0)


def _sc_gather_reduce(table, idx, ww):
    mesh = plsc.VectorSubcoreMesh(core_axis_name="c", subcore_axis_name="s")
    f = pl.kernel(
        _sc_body,
        out_type=jax.ShapeDtypeStruct((_ROWS, _D), jnp.float32),
        mesh=mesh,
        scratch_types=[
            pltpu.VMEM((_RC, _NCOL), jnp.int32),
            pltpu.VMEM((_RC, _NCOL), jnp.float32),
            pltpu.VMEM((_NG * 128, _DH // 2), jnp.int32),
            pltpu.VMEM((_RC, _D), jnp.float32),
            pltpu.SemaphoreType.DMA((_RC,)),
        ],
        compiler_params=pltpu.CompilerParams(needs_layout_passes=False,
                                             use_tc_tiling_on_sc=False),
    )
    return f(table, idx, ww)


def _out_projection(x2, W_out, b_out):
    return pl.pallas_call(
        _vproj_body,
        grid=(_ROWS // _RT,),
        in_specs=[
            pl.BlockSpec((_RT, _D), lambda i: (i, 0)),
            pl.BlockSpec((_D, _D), lambda i: (0, 0)),
            pl.BlockSpec((1, _D), lambda i: (0, 0)),
        ],
        out_specs=pl.BlockSpec((_RT, _D), lambda i: (i, 0)),
        out_shape=jax.ShapeDtypeStruct((_ROWS, _D), jnp.float32),
    )(x2, W_out, b_out.reshape(1, _D))


def _perms():
    lam = np.arange(_LANES)
    h = lam % _H
    lp = lam // _H
    l = lp // _P
    p = lp % _P
    base = (h * _L + l) * _P + p
    perm_x = base * 2
    perm_y = base * 2 + 1
    perm_a = base
    return perm_x, perm_y, perm_a


_PERM_X, _PERM_Y, _PERM_A = _perms()
_G_NP = (np.arange(_LANES)[:, None] % _H == np.arange(_H)[None, :])
_G_NP = _G_NP.astype(np.float32)


def kernel(hidden_states, encoder_hidden_states, reference_points,
           W_off, b_off, W_attn, b_attn, W_val, b_val, W_out, b_out):
    enc2 = encoder_hidden_states.reshape(_B * _S, _D)
    hs2 = hidden_states.reshape(_ROWS, _D)
    rp2 = reference_points.reshape(_ROWS, 4)

    W_cat = jnp.concatenate(
        [W_off[:, _PERM_X], W_off[:, _PERM_Y], W_attn[:, _PERM_A]], axis=1)
    b_cat = jnp.concatenate(
        [b_off[_PERM_X], b_off[_PERM_Y], b_attn[_PERM_A]]).reshape(1, -1)
    G = jnp.asarray(_G_NP)
    GT = jnp.asarray(_G_NP.T)

    value_bf = _value_projection(enc2, W_val, b_val)
    packed = jax.lax.bitcast_convert_type(
        value_bf.reshape(_B * _S, _D // 2, 2), jnp.int32)
    table = packed.reshape(_TROWS, _DH // 2)

    idx, ww = _sampling_params(hs2, rp2, W_cat, b_cat, G, GT)

    x2 = _sc_gather_reduce(table, idx, ww)

    out = _out_projection(x2, W_out, b_out)
    return out.reshape(_B, _Q, _D)
